# baseline (device time: 26927 ns/iter reference)
import jax
import jax.numpy as jnp
from jax import lax
from jax.experimental import pallas as pl
from jax.experimental.pallas import tpu as pltpu

N_DEV = 4
B = 2
SQ = 128
SKV = 128
H_LOC = 4
DH = 64
CTX_COLS = H_LOC * DH
ROWS = B * SQ
D_MODEL = 512
D_QK = 1024


def kernel(x, Wq, K_ext, V_ext, Wo):
    def body(x_ref, wq_ref, k_ref, v_ref, wo_ref, out_ref,
             comm_ref, send_sems, recv_sems):
        my_pos = lax.axis_index("i")
        left = (my_pos - 1) % N_DEV
        right = (my_pos + 1) % N_DEV

        barrier_sem = pltpu.get_barrier_semaphore()
        for nbr in (left, right):
            pl.semaphore_signal(
                barrier_sem, inc=1,
                device_id=(nbr,), device_id_type=pl.DeviceIdType.MESH,
            )
        pl.semaphore_wait(barrier_sem, 2)

        x2d = x_ref[...].reshape(ROWS, D_MODEL)
        wq_loc = wq_ref[:, pl.ds(my_pos * CTX_COLS, CTX_COLS)]
        q2d = jnp.dot(x2d, wq_loc, preferred_element_type=jnp.float32)

        qb = lax.broadcasted_iota(jnp.int32, (SQ, SKV), 0) // 64
        kb = lax.broadcasted_iota(jnp.int32, (SQ, SKV), 1) // 64
        mask = qb == kb

        for b in range(B):
            for h in range(H_LOC):
                q = q2d[b * SQ:(b + 1) * SQ, h * DH:(h + 1) * DH]
                k = k_ref[b, :, h, :]
                v = v_ref[b, :, h, :]
                s = jax.lax.dot_general(
                    q, k, (((1,), (1,)), ((), ())),
                    preferred_element_type=jnp.float32,
                ) * 0.125
                s = jnp.where(mask, s, -1e9)
                m = jnp.max(s, axis=-1, keepdims=True)
                w = jnp.exp(s - m)
                w = w / jnp.sum(w, axis=-1, keepdims=True)
                ctx = jnp.dot(w, v, preferred_element_type=jnp.float32)
                comm_ref[0, b * SQ:(b + 1) * SQ, h * DH:(h + 1) * DH] = ctx

        wo_loc = wo_ref[pl.ds(my_pos * CTX_COLS, CTX_COLS), :]
        acc = jnp.dot(comm_ref[0], wo_loc, preferred_element_type=jnp.float32)

        for h in range(N_DEV - 1):
            rdma = pltpu.make_async_remote_copy(
                src_ref=comm_ref.at[h],
                dst_ref=comm_ref.at[h + 1],
                send_sem=send_sems.at[h],
                recv_sem=recv_sems.at[h],
                device_id=(right,),
                device_id_type=pl.DeviceIdType.MESH,
            )
            rdma.start()
            rdma.wait()
            origin = (my_pos - h - 1) % N_DEV
            wo_j = wo_ref[pl.ds(origin * CTX_COLS, CTX_COLS), :]
            acc = acc + jnp.dot(
                comm_ref[h + 1], wo_j, preferred_element_type=jnp.float32
            )

        out_ref[...] = acc.reshape(B, SQ, D_MODEL)

    return pl.pallas_call(
        body,
        out_shape=jax.ShapeDtypeStruct((B, SQ, D_MODEL), jnp.float32),
        in_specs=[pl.BlockSpec(memory_space=pltpu.VMEM)] * 5,
        out_specs=pl.BlockSpec(memory_space=pltpu.VMEM),
        scratch_shapes=[
            pltpu.VMEM((N_DEV, ROWS, CTX_COLS), jnp.float32),
            pltpu.SemaphoreType.DMA((N_DEV - 1,)),
            pltpu.SemaphoreType.DMA((N_DEV - 1,)),
        ],
        compiler_params=pltpu.CompilerParams(collective_id=0),
    )(x, Wq, K_ext, V_ext, Wo)


# device time: 20799 ns/iter; 1.2946x vs baseline; 1.2946x over previous
import jax
import jax.numpy as jnp
from jax import lax
from jax.experimental import pallas as pl
from jax.experimental.pallas import tpu as pltpu

N_DEV = 4
B = 2
SQ = 128
SKV = 128
H_LOC = 4
DH = 64
CTX_COLS = H_LOC * DH
ROWS = B * SQ
D_MODEL = 512
D_QK = 1024


def kernel(x, Wq, K_ext, V_ext, Wo):
    def body(x_ref, wq_ref, k_ref, v_ref, wo_ref, out_ref,
             comm_ref, send_sems, recv_sems):
        my_pos = lax.axis_index("i")

        barrier_sem = pltpu.get_barrier_semaphore()
        for d in range(1, N_DEV):
            pl.semaphore_signal(
                barrier_sem, inc=1,
                device_id=((my_pos + d) % N_DEV,),
                device_id_type=pl.DeviceIdType.MESH,
            )
        pl.semaphore_wait(barrier_sem, N_DEV - 1)

        x2d = x_ref[...].reshape(ROWS, D_MODEL)
        wq_loc = wq_ref[:, pl.ds(my_pos * CTX_COLS, CTX_COLS)]
        q2d = jnp.dot(x2d, wq_loc, preferred_element_type=jnp.float32)

        qb = lax.broadcasted_iota(jnp.int32, (SQ, SKV), 0) // 64
        kb = lax.broadcasted_iota(jnp.int32, (SQ, SKV), 1) // 64
        mask = qb == kb

        for b in range(B):
            for h in range(H_LOC):
                q = q2d[b * SQ:(b + 1) * SQ, h * DH:(h + 1) * DH]
                k = k_ref[b, :, h, :]
                v = v_ref[b, :, h, :]
                s = jax.lax.dot_general(
                    q, k, (((1,), (1,)), ((), ())),
                    preferred_element_type=jnp.float32,
                ) * 0.125
                s = jnp.where(mask, s, -1e9)
                m = jnp.max(s, axis=-1, keepdims=True)
                w = jnp.exp(s - m)
                w = w / jnp.sum(w, axis=-1, keepdims=True)
                ctx = jnp.dot(w, v, preferred_element_type=jnp.float32)
                comm_ref[0, b * SQ:(b + 1) * SQ, h * DH:(h + 1) * DH] = ctx

        rdma_by_d = {}
        for d in (1, 3, 2):
            rdma = pltpu.make_async_remote_copy(
                src_ref=comm_ref.at[0],
                dst_ref=comm_ref.at[d],
                send_sem=send_sems.at[d - 1],
                recv_sem=recv_sems.at[d - 1],
                device_id=((my_pos + d) % N_DEV,),
                device_id_type=pl.DeviceIdType.MESH,
            )
            rdma.start()
            rdma_by_d[d] = rdma

        wo_loc = wo_ref[pl.ds(my_pos * CTX_COLS, CTX_COLS), :]
        acc = jnp.dot(comm_ref[0], wo_loc, preferred_element_type=jnp.float32)

        for r in (1, 3, 2):
            rdma_by_d[r].wait_recv()
            origin = (my_pos - r) % N_DEV
            wo_j = wo_ref[pl.ds(origin * CTX_COLS, CTX_COLS), :]
            acc = acc + jnp.dot(
                comm_ref[r], wo_j, preferred_element_type=jnp.float32
            )

        for d in (1, 3, 2):
            rdma_by_d[d].wait_send()

        out_ref[...] = acc.reshape(B, SQ, D_MODEL)

    return pl.pallas_call(
        body,
        out_shape=jax.ShapeDtypeStruct((B, SQ, D_MODEL), jnp.float32),
        in_specs=[pl.BlockSpec(memory_space=pltpu.VMEM)] * 5,
        out_specs=pl.BlockSpec(memory_space=pltpu.VMEM),
        scratch_shapes=[
            pltpu.VMEM((N_DEV, ROWS, CTX_COLS), jnp.float32),
            pltpu.SemaphoreType.DMA((N_DEV - 1,)),
            pltpu.SemaphoreType.DMA((N_DEV - 1,)),
        ],
        compiler_params=pltpu.CompilerParams(collective_id=0),
    )(x, Wq, K_ext, V_ext, Wo)


# device time: 18344 ns/iter; 1.4679x vs baseline; 1.1338x over previous
import jax
import jax.numpy as jnp
from jax import lax
from jax.experimental import pallas as pl
from jax.experimental.pallas import tpu as pltpu

N_DEV = 4
B = 2
SQ = 128
SKV = 128
H_LOC = 4
DH = 64
CTX_COLS = H_LOC * DH
ROWS = B * SQ
D_MODEL = 512
D_QK = 1024


def kernel(x, Wq, K_ext, V_ext, Wo):
    def body(x_ref, wq_ref, k_ref, v_ref, wo_ref, out_ref,
             comm_ref, send_sems, recv_sems):
        my_pos = lax.axis_index("i")

        barrier_sem = pltpu.get_barrier_semaphore()
        for d in range(1, N_DEV):
            pl.semaphore_signal(
                barrier_sem, inc=1,
                device_id=((my_pos + d) % N_DEV,),
                device_id_type=pl.DeviceIdType.MESH,
            )
        pl.semaphore_wait(barrier_sem, N_DEV - 1)

        x2d = x_ref[...].reshape(ROWS, D_MODEL).astype(jnp.bfloat16)
        wq_loc = wq_ref[:, pl.ds(my_pos * CTX_COLS, CTX_COLS)]
        q2d = jnp.dot(x2d, wq_loc.astype(jnp.bfloat16),
                      preferred_element_type=jnp.float32)
        q2d = (q2d * 0.125).astype(jnp.bfloat16)

        qb = lax.broadcasted_iota(jnp.int32, (SQ, SKV), 0) // 64
        kb = lax.broadcasted_iota(jnp.int32, (SQ, SKV), 1) // 64
        mask = qb == kb

        for b in range(B):
            for h in range(H_LOC):
                q = q2d[b * SQ:(b + 1) * SQ, h * DH:(h + 1) * DH]
                k = k_ref[b, :, h, :].astype(jnp.bfloat16)
                v = v_ref[b, :, h, :].astype(jnp.bfloat16)
                s = jax.lax.dot_general(
                    q, k, (((1,), (1,)), ((), ())),
                    preferred_element_type=jnp.float32,
                )
                s = jnp.where(mask, s, -1e9)
                m = jnp.max(s, axis=-1, keepdims=True)
                w = jnp.exp(s - m)
                w = (w / jnp.sum(w, axis=-1, keepdims=True)).astype(jnp.bfloat16)
                ctx = jnp.dot(w, v, preferred_element_type=jnp.float32)
                comm_ref[0, b * SQ:(b + 1) * SQ, h * DH:(h + 1) * DH] = (
                    ctx.astype(jnp.bfloat16)
                )

        rdma_by_d = {}
        for d in (1, 3, 2):
            rdma = pltpu.make_async_remote_copy(
                src_ref=comm_ref.at[0],
                dst_ref=comm_ref.at[d],
                send_sem=send_sems.at[d - 1],
                recv_sem=recv_sems.at[d - 1],
                device_id=((my_pos + d) % N_DEV,),
                device_id_type=pl.DeviceIdType.MESH,
            )
            rdma.start()
            rdma_by_d[d] = rdma

        wo_loc = wo_ref[pl.ds(my_pos * CTX_COLS, CTX_COLS), :]
        acc = jnp.dot(comm_ref[0], wo_loc.astype(jnp.bfloat16),
                      preferred_element_type=jnp.float32)

        for r in (1, 3, 2):
            rdma_by_d[r].wait_recv()
            origin = (my_pos - r) % N_DEV
            wo_j = wo_ref[pl.ds(origin * CTX_COLS, CTX_COLS), :]
            acc = acc + jnp.dot(
                comm_ref[r], wo_j.astype(jnp.bfloat16),
                preferred_element_type=jnp.float32,
            )

        for d in (1, 3, 2):
            rdma_by_d[d].wait_send()

        out_ref[...] = acc.reshape(B, SQ, D_MODEL)

    return pl.pallas_call(
        body,
        out_shape=jax.ShapeDtypeStruct((B, SQ, D_MODEL), jnp.float32),
        in_specs=[pl.BlockSpec(memory_space=pltpu.VMEM)] * 5,
        out_specs=pl.BlockSpec(memory_space=pltpu.VMEM),
        scratch_shapes=[
            pltpu.VMEM((N_DEV, ROWS, CTX_COLS), jnp.bfloat16),
            pltpu.SemaphoreType.DMA((N_DEV - 1,)),
            pltpu.SemaphoreType.DMA((N_DEV - 1,)),
        ],
        compiler_params=pltpu.CompilerParams(collective_id=0),
    )(x, Wq, K_ext, V_ext, Wo)


# device time: 13550 ns/iter; 1.9872x vs baseline; 1.3538x over previous
import jax
import jax.numpy as jnp
from jax import lax
from jax.experimental import pallas as pl
from jax.experimental.pallas import tpu as pltpu

N_DEV = 4
B = 2
SQ = 128
SKV = 128
H_LOC = 4
DH = 64
CTX_COLS = H_LOC * DH
ROWS = B * SQ
D_MODEL = 512
D_QK = 1024


def kernel(x, Wq, K_ext, V_ext, Wo):
    my = lax.axis_index("i")
    x_bf = x.astype(jnp.bfloat16)
    wq_bf = (lax.dynamic_slice(Wq, (0, my * CTX_COLS), (D_MODEL, CTX_COLS))
             * 0.125).astype(jnp.bfloat16)
    kt_bf = jnp.transpose(K_ext, (0, 2, 3, 1)).astype(jnp.bfloat16)
    vt_bf = jnp.transpose(V_ext, (0, 2, 3, 1)).astype(jnp.bfloat16)
    wo_bf = Wo.astype(jnp.bfloat16)

    def body(x_ref, wq_ref, k_ref, v_ref, wo_ref, out_ref,
             comm_ref, send_sems, recv_sems):
        my_pos = lax.axis_index("i")

        barrier_sem = pltpu.get_barrier_semaphore()
        for d in range(1, N_DEV):
            pl.semaphore_signal(
                barrier_sem, inc=1,
                device_id=((my_pos + d) % N_DEV,),
                device_id_type=pl.DeviceIdType.MESH,
            )

        x2d = x_ref[...].reshape(ROWS, D_MODEL)
        q2d = jnp.dot(x2d, wq_ref[...],
                      preferred_element_type=jnp.float32
                      ).astype(jnp.bfloat16)

        qb = lax.broadcasted_iota(jnp.int32, (SQ, SKV), 0) // 64
        kb = lax.broadcasted_iota(jnp.int32, (SQ, SKV), 1) // 64
        mask = qb == kb

        rdmas = []
        for b in range(B):
            for h in range(H_LOC):
                q = q2d[b * SQ:(b + 1) * SQ, h * DH:(h + 1) * DH]
                k = k_ref[b, h]
                v = v_ref[b, h]
                s = jax.lax.dot_general(
                    q, k, (((1,), (0,)), ((), ())),
                    preferred_element_type=jnp.float32,
                )
                e = jnp.exp(jnp.where(mask, s, -1e9))
                denom = jnp.sum(e, axis=-1, keepdims=True)
                ctx = jax.lax.dot_general(
                    e.astype(jnp.bfloat16), v, (((1,), (1,)), ((), ())),
                    preferred_element_type=jnp.float32,
                )
                ctx = ctx * (1.0 / denom)
                comm_ref[0, b * SQ:(b + 1) * SQ, h * DH:(h + 1) * DH] = (
                    ctx.astype(jnp.bfloat16)
                )
            if b == 0:
                pl.semaphore_wait(barrier_sem, N_DEV - 1)
            for d in (1, 3, 2):
                rdma = pltpu.make_async_remote_copy(
                    src_ref=comm_ref.at[0, b * SQ:(b + 1) * SQ],
                    dst_ref=comm_ref.at[d, b * SQ:(b + 1) * SQ],
                    send_sem=send_sems.at[d - 1, b],
                    recv_sem=recv_sems.at[d - 1, b],
                    device_id=((my_pos + d) % N_DEV,),
                    device_id_type=pl.DeviceIdType.MESH,
                )
                rdma.start()
                rdmas.append(rdma)

        wo_loc = wo_ref[pl.ds(my_pos * CTX_COLS, CTX_COLS), :]
        acc = jnp.dot(comm_ref[0], wo_loc,
                      preferred_element_type=jnp.float32)

        for r in (1, 3, 2):
            rdmas[r - 1].wait_recv()
            rdmas[2 + r].wait_recv()
            origin = (my_pos - r) % N_DEV
            wo_j = wo_ref[pl.ds(origin * CTX_COLS, CTX_COLS), :]
            acc = acc + jnp.dot(
                comm_ref[r], wo_j,
                preferred_element_type=jnp.float32,
            )

        for rdma in rdmas:
            rdma.wait_send()

        out_ref[...] = acc.astype(jnp.bfloat16).reshape(B, SQ, D_MODEL)

    return pl.pallas_call(
        body,
        out_shape=jax.ShapeDtypeStruct((B, SQ, D_MODEL), jnp.bfloat16),
        in_specs=[pl.BlockSpec(memory_space=pltpu.VMEM)] * 5,
        out_specs=pl.BlockSpec(memory_space=pltpu.VMEM),
        scratch_shapes=[
            pltpu.VMEM((N_DEV, ROWS, CTX_COLS), jnp.bfloat16),
            pltpu.SemaphoreType.DMA((N_DEV - 1, B)),
            pltpu.SemaphoreType.DMA((N_DEV - 1, B)),
        ],
        compiler_params=pltpu.CompilerParams(collective_id=0),
    )(x_bf, wq_bf, kt_bf, vt_bf, wo_bf)
